# baseline (speedup 1.0000x reference)
import functools

import jax
import jax.numpy as jnp
from jax import lax
from jax.experimental import pallas as pl
from jax.experimental.pallas import tpu as pltpu

C1 = 384
C2 = 192
HM = 28
HG = 32


def _fold_sum(v):
    r = v.shape[1] // 8
    t = v.reshape(HM, r, 8, C1)
    return jnp.sum(jnp.sum(jnp.sum(t, axis=0), axis=0), axis=0)


def _mean_select_body(x0_ref, x1_ref, w1_ref, w2_ref, p_ref, s_ref,
                      *, nb, nj, inv_hw):
    bi = pl.program_id(0)
    j = pl.program_id(1)
    p_ref[bi * nj + j, :] = _fold_sum(x0_ref[0]) + _fold_sum(x1_ref[0])

    @pl.when((bi == nb - 1) & (j == nj - 1))
    def _():
        p = p_ref[...].reshape(nb, nj, C1)
        a = p[:, 0:4] + p[:, 4:8]
        a = a[:, 0:2] + a[:, 2:4]
        m = (a[:, 0] + a[:, 1]) * inv_hw
        y1 = lax.dot_general(m, w1_ref[...], (((1,), (1,)), ((), ())),
                             preferred_element_type=jnp.float32)
        y1 = jnp.maximum(y1, 0.0)
        z = lax.dot_general(y1, w2_ref[...], (((1,), (1,)), ((), ())),
                            preferred_element_type=jnp.float32)
        z = 1.0 / (1.0 + jnp.exp(-z))
        ii = lax.broadcasted_iota(jnp.int32, (nb, C1, C1), 1)
        jj = lax.broadcasted_iota(jnp.int32, (nb, C1, C1), 2)
        zi = z[:, :, None]
        zj = z[:, None, :]
        before = (zj > zi) | ((zj == zi) & (jj < ii))
        rank = jnp.sum(before.astype(jnp.int32), axis=2)
        rr = lax.broadcasted_iota(jnp.int32, (nb, C1, C2), 2)
        s_ref[...] = (rank[:, :, None] == rr).astype(jnp.float32)


def _gather_body(x_ref, s_ref, o_ref):
    s = s_ref[0]
    for r in range(HG):
        xr = x_ref[0, r]
        o_ref[0, :, r, :] = lax.dot_general(
            s, xr, (((0,), (1,)), ((), ())),
            preferred_element_type=jnp.float32,
            precision=lax.Precision.HIGHEST)


@jax.jit
def kernel(x, W1, W2):
    b, c, h, w = x.shape
    hw = h * w
    xt = jnp.transpose(x, (0, 2, 3, 1))
    nj = h // HM
    wh = w // 2

    _, sel = pl.pallas_call(
        functools.partial(_mean_select_body, nb=b, nj=nj, inv_hw=1.0 / hw),
        grid=(b, nj),
        in_specs=[
            pl.BlockSpec((1, HM, wh, c), lambda bi, j: (bi, j, 0, 0)),
            pl.BlockSpec((1, HM, wh, c), lambda bi, j: (bi, j, 1, 0)),
            pl.BlockSpec((24, C1), lambda bi, j: (0, 0)),
            pl.BlockSpec((C1, 24), lambda bi, j: (0, 0)),
        ],
        out_specs=[
            pl.BlockSpec((b * nj, c), lambda bi, j: (0, 0)),
            pl.BlockSpec((b, C1, C2), lambda bi, j: (0, 0, 0)),
        ],
        out_shape=[
            jax.ShapeDtypeStruct((b * nj, c), jnp.float32),
            jax.ShapeDtypeStruct((b, C1, C2), jnp.float32),
        ],
    )(xt, xt, W1, W2)

    out = pl.pallas_call(
        _gather_body,
        grid=(b, h // HG),
        in_specs=[pl.BlockSpec((1, HG, w, c), lambda bi, j: (bi, j, 0, 0)),
                  pl.BlockSpec((1, C1, C2), lambda bi, j: (bi, 0, 0))],
        out_specs=pl.BlockSpec((1, C2, HG, w), lambda bi, j: (bi, 0, j, 0)),
        out_shape=jax.ShapeDtypeStruct((b, C2, h, w), jnp.float32),
    )(xt, sel)
    return out


# speedup vs baseline: 1.0008x; 1.0008x over previous
import functools

import jax
import jax.numpy as jnp
from jax import lax
from jax.experimental import pallas as pl
from jax.experimental.pallas import tpu as pltpu

C1 = 384
C2 = 192
HM = 28
HG = 16


def _fold_sum(v):
    r = v.shape[1] // 8
    t = v.reshape(HM, r, 8, C1)
    return jnp.sum(jnp.sum(jnp.sum(t, axis=0), axis=0), axis=0)


def _mean_select_body(x0_ref, x1_ref, w1_ref, w2_ref, p_ref, s_ref,
                      *, nb, nj, inv_hw):
    bi = pl.program_id(0)
    j = pl.program_id(1)
    p_ref[bi * nj + j, :] = _fold_sum(x0_ref[0]) + _fold_sum(x1_ref[0])

    @pl.when((bi == nb - 1) & (j == nj - 1))
    def _():
        p = p_ref[...].reshape(nb, nj, C1)
        a = p[:, 0:4] + p[:, 4:8]
        a = a[:, 0:2] + a[:, 2:4]
        m = (a[:, 0] + a[:, 1]) * inv_hw
        y1 = lax.dot_general(m, w1_ref[...], (((1,), (1,)), ((), ())),
                             preferred_element_type=jnp.float32)
        y1 = jnp.maximum(y1, 0.0)
        z = lax.dot_general(y1, w2_ref[...], (((1,), (1,)), ((), ())),
                            preferred_element_type=jnp.float32)
        z = 1.0 / (1.0 + jnp.exp(-z))
        ii = lax.broadcasted_iota(jnp.int32, (nb, C1, C1), 1)
        jj = lax.broadcasted_iota(jnp.int32, (nb, C1, C1), 2)
        zi = z[:, :, None]
        zj = z[:, None, :]
        before = (zj > zi) | ((zj == zi) & (jj < ii))
        rank = jnp.sum(before.astype(jnp.int32), axis=2)
        rr = lax.broadcasted_iota(jnp.int32, (nb, C1, C2), 2)
        s_ref[...] = (rank[:, :, None] == rr).astype(jnp.float32)


def _gather_body(x0_ref, x1_ref, s_ref, o_ref):
    s = s_ref[0]
    hh = HG // 2
    for r in range(HG):
        xr = (x0_ref if r < hh else x1_ref)[0, r % hh]
        o_ref[0, :, r, :] = lax.dot_general(
            s, xr, (((0,), (1,)), ((), ())),
            preferred_element_type=jnp.float32,
            precision=lax.Precision.HIGHEST)


@jax.jit
def kernel(x, W1, W2):
    b, c, h, w = x.shape
    hw = h * w
    xt = jnp.transpose(x, (0, 2, 3, 1))
    nj = h // HM
    wh = w // 2

    _, sel = pl.pallas_call(
        functools.partial(_mean_select_body, nb=b, nj=nj, inv_hw=1.0 / hw),
        grid=(b, nj),
        in_specs=[
            pl.BlockSpec((1, HM, wh, c), lambda bi, j: (bi, j, 0, 0)),
            pl.BlockSpec((1, HM, wh, c), lambda bi, j: (bi, j, 1, 0)),
            pl.BlockSpec((24, C1), lambda bi, j: (0, 0)),
            pl.BlockSpec((C1, 24), lambda bi, j: (0, 0)),
        ],
        out_specs=[
            pl.BlockSpec((b * nj, c), lambda bi, j: (0, 0)),
            pl.BlockSpec((b, C1, C2), lambda bi, j: (0, 0, 0)),
        ],
        out_shape=[
            jax.ShapeDtypeStruct((b * nj, c), jnp.float32),
            jax.ShapeDtypeStruct((b, C1, C2), jnp.float32),
        ],
    )(xt, xt, W1, W2)

    out = pl.pallas_call(
        _gather_body,
        grid=(b, h // HG),
        in_specs=[pl.BlockSpec((1, HG // 2, w, c),
                               lambda bi, j: (bi, 2 * j, 0, 0)),
                  pl.BlockSpec((1, HG // 2, w, c),
                               lambda bi, j: (bi, 2 * j + 1, 0, 0)),
                  pl.BlockSpec((1, C1, C2), lambda bi, j: (bi, 0, 0))],
        out_specs=pl.BlockSpec((1, C2, HG, w), lambda bi, j: (bi, 0, j, 0)),
        out_shape=jax.ShapeDtypeStruct((b, C2, h, w), jnp.float32),
    )(xt, xt, sel)
    return out


# speedup vs baseline: 1.0037x; 1.0028x over previous
import functools

import jax
import jax.numpy as jnp
from jax import lax
from jax.experimental import pallas as pl
from jax.experimental.pallas import tpu as pltpu

C1 = 384
C2 = 192
HM = 28
HG = 16


def _fold_sum(v):
    r = v.shape[1] // 8
    t = v.reshape(HM, r, 8, C1)
    return jnp.sum(jnp.sum(jnp.sum(t, axis=0), axis=0), axis=0)


def _mean_select_body(x0_ref, x1_ref, w1_ref, w2_ref, p_ref, s_ref,
                      *, nb, nj, inv_hw):
    bi = pl.program_id(0)
    j = pl.program_id(1)
    p_ref[bi * nj + j, :] = _fold_sum(x0_ref[0]) + _fold_sum(x1_ref[0])

    @pl.when((bi == nb - 1) & (j == nj - 1))
    def _():
        p = p_ref[...].reshape(nb, nj, C1)
        a = p[:, 0:4] + p[:, 4:8]
        a = a[:, 0:2] + a[:, 2:4]
        m = (a[:, 0] + a[:, 1]) * inv_hw
        y1 = lax.dot_general(m, w1_ref[...], (((1,), (1,)), ((), ())),
                             preferred_element_type=jnp.float32)
        y1 = jnp.maximum(y1, 0.0)
        z = lax.dot_general(y1, w2_ref[...], (((1,), (1,)), ((), ())),
                            preferred_element_type=jnp.float32)
        z = 1.0 / (1.0 + jnp.exp(-z))
        ii = lax.broadcasted_iota(jnp.int32, (nb, C1, C1), 1)
        jj = lax.broadcasted_iota(jnp.int32, (nb, C1, C1), 2)
        zi = z[:, :, None]
        zj = z[:, None, :]
        before = (zj > zi) | ((zj == zi) & (jj < ii))
        rank = jnp.sum(before.astype(jnp.int32), axis=2)
        rr = lax.broadcasted_iota(jnp.int32, (nb, C1, C2), 2)
        s_ref[...] = (rank[:, :, None] == rr).astype(jnp.float32)


def _gather_body(x_ref, s_ref, o_ref):
    s = s_ref[0]
    for r in range(HG):
        xr = x_ref[0, r]
        o_ref[0, :, r, :] = lax.dot_general(
            s, xr, (((0,), (1,)), ((), ())),
            preferred_element_type=jnp.float32,
            precision=lax.Precision.HIGHEST)


@jax.jit
def kernel(x, W1, W2):
    b, c, h, w = x.shape
    hw = h * w
    xt = jnp.transpose(x, (0, 2, 3, 1))
    nj = h // HM
    wh = w // 2

    _, sel = pl.pallas_call(
        functools.partial(_mean_select_body, nb=b, nj=nj, inv_hw=1.0 / hw),
        grid=(b, nj),
        in_specs=[
            pl.BlockSpec((1, HM, wh, c), lambda bi, j: (bi, j, 0, 0)),
            pl.BlockSpec((1, HM, wh, c), lambda bi, j: (bi, j, 1, 0)),
            pl.BlockSpec((24, C1), lambda bi, j: (0, 0)),
            pl.BlockSpec((C1, 24), lambda bi, j: (0, 0)),
        ],
        out_specs=[
            pl.BlockSpec((b * nj, c), lambda bi, j: (0, 0)),
            pl.BlockSpec((b, C1, C2), lambda bi, j: (0, 0, 0)),
        ],
        out_shape=[
            jax.ShapeDtypeStruct((b * nj, c), jnp.float32),
            jax.ShapeDtypeStruct((b, C1, C2), jnp.float32),
        ],
    )(xt, xt, W1, W2)

    out = pl.pallas_call(
        _gather_body,
        grid=(b, h // HG),
        in_specs=[pl.BlockSpec((1, HG, w, c), lambda bi, j: (bi, j, 0, 0)),
                  pl.BlockSpec((1, C1, C2), lambda bi, j: (bi, 0, 0))],
        out_specs=pl.BlockSpec((1, C2, HG, w), lambda bi, j: (bi, 0, j, 0)),
        out_shape=jax.ShapeDtypeStruct((b, C2, h, w), jnp.float32),
    )(xt, sel)
    return out
